# baseline (speedup 1.0000x reference)
import jax
import jax.numpy as jnp
from jax import lax
from jax.experimental import pallas as pl
from jax.experimental.pallas import tpu as pltpu
from jax.experimental.pallas import tpu_sc as plsc

_NC = 2
_NS = 16
_NW = _NC * _NS
_L = 16

_BN_EPS = 1e-5
_TM = 512
_V = 16
_NBUF = 3
_CH0_SHARE = 0.5


def _split_chunks(total_chunks):
    nch0 = int(round(total_chunks * _CH0_SHARE / (_NS * _NBUF))) * _NBUF
    nch1 = total_chunks // _NS - nch0
    return nch0, nch1


def _gather_sum7_sc(table, idxw, n_out, nch0, nch1):
    T, C = table.shape
    CL = C // _L

    mesh = plsc.VectorSubcoreMesh(
        core_axis_name="c", subcore_axis_name="s",
        num_cores=_NC, num_subcores=_NS)

    def body(tab_hbm, idx_hbm, out_hbm,
             idx_all, r0, r1, r2, a0, a1, a2, g0, g1, g2, o0, o1, o2):
        rows = (r0, r1, r2)
        acc = (a0, a1, a2)
        semg = (g0, g1, g2)
        semo = (o0, o1, o2)
        c = lax.axis_index("c")
        s = lax.axis_index("s")
        wid = c * _NS + s

        def run(nch, base):
            pltpu.sync_copy(idx_hbm.at[wid], idx_all)
            for b in range(_NBUF):
                pltpu.async_copy(tab_hbm.at[idx_all.at[b]], rows[b], semg[b])

            def group(g, carry):
                for b in range(_NBUF):
                    ci = g * _NBUF + b
                    pltpu.make_async_copy(
                        tab_hbm.at[idx_all.at[ci]], rows[b], semg[b]).wait()

                    @pl.when(g > 0)
                    def _():
                        pltpu.make_async_copy(
                            acc[b],
                            out_hbm.at[pl.ds(base + (ci - _NBUF) * _V, _V)],
                            semo[b]).wait()

                    def per_row(v, c2):
                        for cc in range(CL):
                            sl = pl.ds(cc * _L, _L)
                            sv = rows[b][v * 7, sl]
                            for k in range(1, 7):
                                sv = sv + rows[b][v * 7 + k, sl]
                            acc[b][v, sl] = sv
                        return c2

                    lax.fori_loop(0, _V, per_row, 0)
                    pltpu.async_copy(
                        acc[b], out_hbm.at[pl.ds(base + ci * _V, _V)],
                        semo[b])

                    @pl.when(ci + _NBUF < nch)
                    def _():
                        pltpu.async_copy(
                            tab_hbm.at[idx_all.at[ci + _NBUF]], rows[b],
                            semg[b])
                return carry

            lax.fori_loop(0, nch // _NBUF, group, 0)
            for b in range(_NBUF):
                ci = nch - _NBUF + b
                pltpu.make_async_copy(
                    acc[b], out_hbm.at[pl.ds(base + ci * _V, _V)],
                    semo[b]).wait()

        @pl.when(c == 0)
        def _():
            run(nch0, s * (nch0 * _V))

        @pl.when(c == 1)
        def _():
            run(nch1, (_NS * nch0 + s * nch1) * _V)

    f = pl.kernel(
        body,
        out_type=jax.ShapeDtypeStruct((n_out, C), jnp.float32),
        mesh=mesh,
        scratch_types=(
            [pltpu.VMEM((max(nch0, nch1), _V * 7), jnp.int32)]
            + [pltpu.VMEM((_V * 7, C), jnp.float32)] * 3
            + [pltpu.VMEM((_V, C), jnp.float32)] * 3
            + [pltpu.SemaphoreType.DMA] * 6
        ),
    )
    return f(table, idxw)


def _block_idx_per_worker(idx_flat, nch0, nch1):
    ch = idx_flat.reshape(-1, _V * 7)
    m = max(nch0, nch1)
    top = ch[: _NS * nch0].reshape(_NS, nch0, _V * 7)
    top = jnp.pad(top, ((0, 0), (0, m - nch0), (0, 0)))
    bot = ch[_NS * nch0:].reshape(_NS, nch1, _V * 7)
    bot = jnp.pad(bot, ((0, 0), (0, m - nch1), (0, 0)))
    return jnp.concatenate([top, bot], axis=0)


def _zp_matmul_tc(xp, Wz, scale):
    P, C = xp.shape
    K7 = Wz.shape[1]

    def body(x_ref, w_ref, o_ref):
        o_ref[...] = jnp.dot(x_ref[...] * scale, w_ref[...],
                             preferred_element_type=jnp.float32)

    return pl.pallas_call(
        body,
        grid=(P // _TM,),
        in_specs=[pl.BlockSpec((_TM, C), lambda i: (i, 0)),
                  pl.BlockSpec((C, K7), lambda i: (0, 0))],
        out_specs=pl.BlockSpec((_TM, K7), lambda i: (i, 0)),
        out_shape=jax.ShapeDtypeStruct((P, K7), jnp.float32),
    )(xp, Wz)


def _stats_tc(z, n_valid_per_batch, tiles_per_batch):
    P, C = z.shape

    def body(z_ref, s_ref):
        i = pl.program_id(0)

        @pl.when(i == 0)
        def _():
            s_ref[...] = jnp.zeros_like(s_ref)

        jb = (i % tiles_per_batch) * _TM
        rows = lax.broadcasted_iota(jnp.int32, (_TM, C), 0) + jb
        zm = jnp.where(rows < n_valid_per_batch, z_ref[...], 0.0)
        s_ref[0:1, :] += jnp.sum(zm, axis=0, keepdims=True)
        s_ref[1:2, :] += jnp.sum(zm * zm, axis=0, keepdims=True)

    return pl.pallas_call(
        body,
        grid=(P // _TM,),
        in_specs=[pl.BlockSpec((_TM, C), lambda i: (i, 0))],
        out_specs=pl.BlockSpec((8, C), lambda i: (0, 0)),
        out_shape=jax.ShapeDtypeStruct((8, C), jnp.float32),
    )(z)


def _final_tc(z, x1, stats, params, pcol, Wa, Wb, n_valid, B, Nl, tpb):
    P, C = z.shape
    inv_n = 1.0 / float(n_valid)

    def body(z_ref, x1_ref, s_ref, p_ref, pc_ref, wa_ref, wb_ref, o_ref):
        mean = s_ref[0:1, :] * inv_n
        var = s_ref[1:2, :] * inv_n - mean * mean
        sc = p_ref[0:1, :] * lax.rsqrt(var + _BN_EPS)
        tr = p_ref[1:2, :] - mean * sc
        zn = z_ref[...] * sc + tr
        zn = jnp.where(zn >= 0, zn, 0.2 * zn)
        y = lax.dot_general(wa_ref[...], zn, (((1,), (1,)), ((), ())),
                            preferred_element_type=jnp.float32)
        y = y + jnp.dot(wb_ref[...], x1_ref[0],
                        preferred_element_type=jnp.float32)
        o_ref[0] = y + pc_ref[:, 0:1]

    return pl.pallas_call(
        body,
        grid=(B, tpb),
        in_specs=[pl.BlockSpec((_TM, C), lambda b, t: (b * tpb + t, 0)),
                  pl.BlockSpec((1, C, _TM), lambda b, t: (b, 0, t)),
                  pl.BlockSpec((8, C), lambda b, t: (0, 0)),
                  pl.BlockSpec((8, C), lambda b, t: (0, 0)),
                  pl.BlockSpec((C, 8), lambda b, t: (0, 0)),
                  pl.BlockSpec((C, C), lambda b, t: (0, 0)),
                  pl.BlockSpec((C, C), lambda b, t: (0, 0))],
        out_specs=pl.BlockSpec((1, C, _TM), lambda b, t: (b, 0, t)),
        out_shape=jax.ShapeDtypeStruct((B, C, Nl), jnp.float32),
    )(z, x1, stats, params, pcol, Wa, Wb)


def kernel(x, x1, neigh_orders, pool_neigh_orders, W1, b1, gamma, beta, Wc, bc):
    B, C, Nh = x.shape
    CO = x1.shape[1]
    Nl = (Nh + 6) // 4
    TPB = (Nl + _TM - 1) // _TM
    Pb = TPB * _TM
    P = B * Pb
    NCH0, NCH1 = _split_chunks(P // _V)

    xT = x.transpose(0, 2, 1).reshape(B * Nh, C)

    pad_bj = ((0, 0), (0, Pb - Nl), (0, 0))
    boffs_h = (jnp.arange(B, dtype=jnp.int32) * Nh)[:, None]
    pool_abs = (pool_neigh_orders[: Nl * 7][None, :] + boffs_h)
    pool_abs = jnp.pad(pool_abs.reshape(B, Nl, 7), pad_bj).reshape(-1)

    boffs_l = (jnp.arange(B, dtype=jnp.int32) * Pb)[:, None]
    k_off = jnp.tile(jnp.arange(7, dtype=jnp.int32), Nl)[None, :]
    ring_abs = (neigh_orders[None, :] + boffs_l) * 7 + k_off
    ring_abs = jnp.pad(ring_abs.reshape(B, Nl, 7), pad_bj).reshape(-1)

    Wz = W1.reshape(CO, 7, C).transpose(2, 1, 0).reshape(C, 7 * CO)
    Wa = Wc[:, :CO]
    Wb = Wc[:, CO:]
    params = jnp.concatenate(
        [gamma[None, :], beta[None, :],
         jnp.zeros((6, CO), jnp.float32)], axis=0)
    pcol = jnp.concatenate(
        [bc[:, None], jnp.zeros((CO, 7), jnp.float32)], axis=1)

    xp = _gather_sum7_sc(xT, _block_idx_per_worker(pool_abs, NCH0, NCH1),
                         P, NCH0, NCH1)
    zp = _zp_matmul_tc(xp, Wz, 1.0 / 7.0)
    z = _gather_sum7_sc(zp.reshape(P * 7, CO),
                        _block_idx_per_worker(ring_abs, NCH0, NCH1),
                        P, NCH0, NCH1)
    stats = _stats_tc(z, Nl, TPB)
    return _final_tc(z, x1, stats, params, pcol, Wa, Wb,
                     B * Nl, B, Nl, TPB)


# speedup vs baseline: 1.6541x; 1.6541x over previous
import jax
import jax.numpy as jnp
from jax import lax
from jax.experimental import pallas as pl
from jax.experimental.pallas import tpu as pltpu
from jax.experimental.pallas import tpu_sc as plsc

_NC = 2
_NS = 16
_NW = _NC * _NS
_L = 16

_BN_EPS = 1e-5
_TM = 512
_V = 16
_NBUF = 3


def _gather_sum7_sc(table, idxw, n_out):
    T, C = table.shape
    CL = C // _L
    nch = idxw.shape[1]
    pw = nch * _V

    mesh = plsc.VectorSubcoreMesh(
        core_axis_name="c", subcore_axis_name="s",
        num_cores=_NC, num_subcores=_NS)

    def body(tab_hbm, idx_hbm, out_hbm,
             idx_all, r0, r1, r2, a0, a1, a2, g0, g1, g2, o0, o1, o2):
        rows = (r0, r1, r2)
        acc = (a0, a1, a2)
        semg = (g0, g1, g2)
        semo = (o0, o1, o2)
        wid = lax.axis_index("c") * _NS + lax.axis_index("s")
        base = wid * pw
        pltpu.sync_copy(idx_hbm.at[wid], idx_all)
        for b in range(_NBUF):
            pltpu.async_copy(tab_hbm.at[idx_all.at[b]], rows[b], semg[b])

        def group(g, carry):
            for b in range(_NBUF):
                ci = g * _NBUF + b
                pltpu.make_async_copy(
                    tab_hbm.at[idx_all.at[ci]], rows[b], semg[b]).wait()

                @pl.when(g > 0)
                def _():
                    pltpu.make_async_copy(
                        acc[b],
                        out_hbm.at[pl.ds(base + (ci - _NBUF) * _V, _V)],
                        semo[b]).wait()

                def per_row(v, c2):
                    for cc in range(CL):
                        sl = pl.ds(cc * _L, _L)
                        sv = rows[b][v * 7, sl]
                        for k in range(1, 7):
                            sv = sv + rows[b][v * 7 + k, sl]
                        acc[b][v, sl] = sv
                    return c2

                lax.fori_loop(0, _V, per_row, 0)
                pltpu.async_copy(
                    acc[b], out_hbm.at[pl.ds(base + ci * _V, _V)], semo[b])

                @pl.when(ci + _NBUF < nch)
                def _():
                    pltpu.async_copy(
                        tab_hbm.at[idx_all.at[ci + _NBUF]], rows[b], semg[b])
            return carry

        lax.fori_loop(0, nch // _NBUF, group, 0)
        for b in range(_NBUF):
            ci = nch - _NBUF + b
            pltpu.make_async_copy(
                acc[b], out_hbm.at[pl.ds(base + ci * _V, _V)], semo[b]).wait()

    f = pl.kernel(
        body,
        out_type=jax.ShapeDtypeStruct((n_out, C), jnp.float32),
        mesh=mesh,
        scratch_types=(
            [pltpu.VMEM((nch, _V * 7), jnp.int32)]
            + [pltpu.VMEM((_V * 7, C), jnp.float32)] * 3
            + [pltpu.VMEM((_V, C), jnp.float32)] * 3
            + [pltpu.SemaphoreType.DMA] * 6
        ),
    )
    return f(table, idxw)


def _zp_matmul_tc(xp, Wz, scale):
    P, C = xp.shape
    K7 = Wz.shape[1]

    def body(x_ref, w_ref, o_ref):
        o_ref[...] = jnp.dot(x_ref[...] * scale, w_ref[...],
                             preferred_element_type=jnp.float32)

    return pl.pallas_call(
        body,
        grid=(P // _TM,),
        in_specs=[pl.BlockSpec((_TM, C), lambda i: (i, 0)),
                  pl.BlockSpec((C, K7), lambda i: (0, 0))],
        out_specs=pl.BlockSpec((_TM, K7), lambda i: (i, 0)),
        out_shape=jax.ShapeDtypeStruct((P, K7), jnp.float32),
    )(xp, Wz)


def _stats_tc(z, n_valid_per_batch, tiles_per_batch):
    P, C = z.shape

    def body(z_ref, s_ref):
        i = pl.program_id(0)

        @pl.when(i == 0)
        def _():
            s_ref[...] = jnp.zeros_like(s_ref)

        jb = (i % tiles_per_batch) * _TM
        rows = lax.broadcasted_iota(jnp.int32, (_TM, C), 0) + jb
        zm = jnp.where(rows < n_valid_per_batch, z_ref[...], 0.0)
        s_ref[0:1, :] += jnp.sum(zm, axis=0, keepdims=True)
        s_ref[1:2, :] += jnp.sum(zm * zm, axis=0, keepdims=True)

    return pl.pallas_call(
        body,
        grid=(P // _TM,),
        in_specs=[pl.BlockSpec((_TM, C), lambda i: (i, 0))],
        out_specs=pl.BlockSpec((8, C), lambda i: (0, 0)),
        out_shape=jax.ShapeDtypeStruct((8, C), jnp.float32),
    )(z)


def _final_tc(z, x1, stats, params, pcol, Wa, Wb, n_valid, B, Nl, tpb):
    P, C = z.shape
    inv_n = 1.0 / float(n_valid)

    def body(z_ref, x1_ref, s_ref, p_ref, pc_ref, wa_ref, wb_ref, o_ref):
        mean = s_ref[0:1, :] * inv_n
        var = s_ref[1:2, :] * inv_n - mean * mean
        sc = p_ref[0:1, :] * lax.rsqrt(var + _BN_EPS)
        tr = p_ref[1:2, :] - mean * sc
        zn = z_ref[...] * sc + tr
        zn = jnp.where(zn >= 0, zn, 0.2 * zn)
        y = lax.dot_general(wa_ref[...], zn, (((1,), (1,)), ((), ())),
                            preferred_element_type=jnp.float32)
        y = y + jnp.dot(wb_ref[...], x1_ref[0],
                        preferred_element_type=jnp.float32)
        o_ref[0] = y + pc_ref[:, 0:1]

    return pl.pallas_call(
        body,
        grid=(B, tpb),
        in_specs=[pl.BlockSpec((_TM, C), lambda b, t: (b * tpb + t, 0)),
                  pl.BlockSpec((1, C, _TM), lambda b, t: (b, 0, t)),
                  pl.BlockSpec((8, C), lambda b, t: (0, 0)),
                  pl.BlockSpec((8, C), lambda b, t: (0, 0)),
                  pl.BlockSpec((C, 8), lambda b, t: (0, 0)),
                  pl.BlockSpec((C, C), lambda b, t: (0, 0)),
                  pl.BlockSpec((C, C), lambda b, t: (0, 0))],
        out_specs=pl.BlockSpec((1, C, _TM), lambda b, t: (b, 0, t)),
        out_shape=jax.ShapeDtypeStruct((B, C, Nl), jnp.float32),
    )(z, x1, stats, params, pcol, Wa, Wb)


def kernel(x, x1, neigh_orders, pool_neigh_orders, W1, b1, gamma, beta, Wc, bc):
    B, C, Nh = x.shape
    CO = x1.shape[1]
    Nl = (Nh + 6) // 4
    TPB = (Nl + _TM - 1) // _TM
    Pb = TPB * _TM
    P = B * Pb

    xT = x.transpose(0, 2, 1).reshape(B * Nh, C)

    pad_j = ((0, Pb - Nl), (0, 0))
    boffs_h = (jnp.arange(B, dtype=jnp.int32) * Nh)[:, None, None]
    pool_pad = jnp.pad(pool_neigh_orders[: Nl * 7].reshape(Nl, 7), pad_j)
    pool_abs = (pool_pad[None] + boffs_h).reshape(-1)

    boffs_l = (jnp.arange(B, dtype=jnp.int32) * (Pb * 7))[:, None, None]
    k_off = jnp.arange(7, dtype=jnp.int32)[None, :]
    ring_pad = jnp.pad(neigh_orders.reshape(Nl, 7) * 7 + k_off, pad_j)
    ring_abs = (ring_pad[None] + boffs_l).reshape(-1)

    Wz = W1.reshape(CO, 7, C).transpose(2, 1, 0).reshape(C, 7 * CO)
    Wa = Wc[:, :CO]
    Wb = Wc[:, CO:]
    params = jnp.concatenate(
        [gamma[None, :], beta[None, :],
         jnp.zeros((6, CO), jnp.float32)], axis=0)
    pcol = jnp.concatenate(
        [bc[:, None], jnp.zeros((CO, 7), jnp.float32)], axis=1)

    idx_shape = (_NW, P // (_V * _NW), _V * 7)
    xp = _gather_sum7_sc(xT, pool_abs.reshape(idx_shape), P)
    zp = _zp_matmul_tc(xp, Wz, 1.0 / 7.0)
    z = _gather_sum7_sc(zp.reshape(P * 7, CO),
                        ring_abs.reshape(idx_shape), P)
    stats = _stats_tc(z, Nl, TPB)
    return _final_tc(z, x1, stats, params, pcol, Wa, Wb,
                     B * Nl, B, Nl, TPB)


# speedup vs baseline: 1.8330x; 1.1082x over previous
import jax
import jax.numpy as jnp
from jax import lax
from jax.experimental import pallas as pl
from jax.experimental.pallas import tpu as pltpu
from jax.experimental.pallas import tpu_sc as plsc

_NC = 2
_NS = 16
_NW = _NC * _NS
_L = 16

_BN_EPS = 1e-5
_TM = 512
_V = 16
_NBUF = 3


def _gather_sum7_sc(table, idxw, n_out):
    T, C = table.shape
    CL = C // _L
    nch = idxw.shape[1]
    pw = nch * _V

    mesh = plsc.VectorSubcoreMesh(
        core_axis_name="c", subcore_axis_name="s",
        num_cores=_NC, num_subcores=_NS)

    def body(tab_hbm, idx_hbm, out_hbm,
             idx_all, r0, r1, r2, a0, a1, a2, g0, g1, g2, o0, o1, o2):
        rows = (r0, r1, r2)
        acc = (a0, a1, a2)
        semg = (g0, g1, g2)
        semo = (o0, o1, o2)
        wid = lax.axis_index("c") * _NS + lax.axis_index("s")
        base = wid * pw
        pltpu.sync_copy(idx_hbm.at[wid], idx_all)
        for b in range(_NBUF):
            pltpu.async_copy(tab_hbm.at[idx_all.at[b]], rows[b], semg[b])

        def group(g, carry):
            for b in range(_NBUF):
                ci = g * _NBUF + b
                pltpu.make_async_copy(
                    tab_hbm.at[idx_all.at[ci]], rows[b], semg[b]).wait()

                @pl.when(g > 0)
                def _():
                    pltpu.make_async_copy(
                        acc[b],
                        out_hbm.at[pl.ds(base + (ci - _NBUF) * _V, _V)],
                        semo[b]).wait()

                def per_row(v, c2):
                    for cc in range(CL):
                        sl = pl.ds(cc * _L, _L)
                        sv = rows[b][v * 7, sl]
                        for k in range(1, 7):
                            sv = sv + rows[b][v * 7 + k, sl]
                        acc[b][v, sl] = sv
                    return c2

                lax.fori_loop(0, _V, per_row, 0)
                pltpu.async_copy(
                    acc[b], out_hbm.at[pl.ds(base + ci * _V, _V)], semo[b])

                @pl.when(ci + _NBUF < nch)
                def _():
                    pltpu.async_copy(
                        tab_hbm.at[idx_all.at[ci + _NBUF]], rows[b], semg[b])
            return carry

        lax.fori_loop(0, nch // _NBUF, group, 0)
        for b in range(_NBUF):
            ci = nch - _NBUF + b
            pltpu.make_async_copy(
                acc[b], out_hbm.at[pl.ds(base + ci * _V, _V)], semo[b]).wait()

    f = pl.kernel(
        body,
        out_type=jax.ShapeDtypeStruct((n_out, C), jnp.float32),
        mesh=mesh,
        scratch_types=(
            [pltpu.VMEM((nch, _V * 7), jnp.int32)]
            + [pltpu.VMEM((_V * 7, C), jnp.float32)] * 3
            + [pltpu.VMEM((_V, C), jnp.float32)] * 3
            + [pltpu.SemaphoreType.DMA] * 6
        ),
    )
    return f(table, idxw)


def _zp_matmul_tc(xp, Wz, scale):
    P, C = xp.shape
    K7 = Wz.shape[1]

    def body(x_ref, w_ref, o_ref):
        o_ref[...] = jnp.dot(x_ref[...] * scale, w_ref[...],
                             preferred_element_type=jnp.float32)

    return pl.pallas_call(
        body,
        grid=(P // _TM,),
        in_specs=[pl.BlockSpec((_TM, C), lambda i: (i, 0)),
                  pl.BlockSpec((C, K7), lambda i: (0, 0))],
        out_specs=pl.BlockSpec((_TM, K7), lambda i: (i, 0)),
        out_shape=jax.ShapeDtypeStruct((P, K7), jnp.float32),
    )(xp, Wz)


def _stats_tc(z, n_valid):
    P, C = z.shape

    def body(z_ref, s_ref):
        i = pl.program_id(0)

        @pl.when(i == 0)
        def _():
            s_ref[...] = jnp.zeros_like(s_ref)

        rows = lax.broadcasted_iota(jnp.int32, (_TM, C), 0) + i * _TM
        zm = jnp.where(rows < n_valid, z_ref[...], 0.0)
        s_ref[0:1, :] += jnp.sum(zm, axis=0, keepdims=True)
        s_ref[1:2, :] += jnp.sum(zm * zm, axis=0, keepdims=True)

    return pl.pallas_call(
        body,
        grid=(P // _TM,),
        in_specs=[pl.BlockSpec((_TM, C), lambda i: (i, 0))],
        out_specs=pl.BlockSpec((8, C), lambda i: (0, 0)),
        out_shape=jax.ShapeDtypeStruct((8, C), jnp.float32),
    )(z)


def _final_tc(z, x1p, stats, params, WaT, WbT, n_valid):
    P, C = z.shape
    inv_n = 1.0 / float(n_valid)

    def body(z_ref, x1_ref, s_ref, p_ref, wa_ref, wb_ref, o_ref):
        mean = s_ref[0:1, :] * inv_n
        var = s_ref[1:2, :] * inv_n - mean * mean
        sc = p_ref[0:1, :] * lax.rsqrt(var + _BN_EPS)
        tr = p_ref[1:2, :] - mean * sc
        zn = z_ref[...] * sc + tr
        zn = jnp.where(zn >= 0, zn, 0.2 * zn)
        acc = jnp.dot(zn, wa_ref[...], preferred_element_type=jnp.float32)
        acc = acc + jnp.dot(x1_ref[...], wb_ref[...],
                            preferred_element_type=jnp.float32)
        o_ref[...] = acc + p_ref[2:3, :]

    return pl.pallas_call(
        body,
        grid=(P // _TM,),
        in_specs=[pl.BlockSpec((_TM, C), lambda i: (i, 0)),
                  pl.BlockSpec((_TM, C), lambda i: (i, 0)),
                  pl.BlockSpec((8, C), lambda i: (0, 0)),
                  pl.BlockSpec((8, C), lambda i: (0, 0)),
                  pl.BlockSpec((C, C), lambda i: (0, 0)),
                  pl.BlockSpec((C, C), lambda i: (0, 0))],
        out_specs=pl.BlockSpec((_TM, C), lambda i: (i, 0)),
        out_shape=jax.ShapeDtypeStruct((P, C), jnp.float32),
    )(z, x1p, stats, params, WaT, WbT)


def kernel(x, x1, neigh_orders, pool_neigh_orders, W1, b1, gamma, beta, Wc, bc):
    B, C, Nh = x.shape
    CO = x1.shape[1]
    Nl = (Nh + 6) // 4
    R = B * Nl
    P = ((R + _NW * _V - 1) // (_NW * _V)) * (_NW * _V)

    xT = x.transpose(0, 2, 1).reshape(B * Nh, C)
    x1p = jnp.pad(x1.transpose(0, 2, 1).reshape(R, CO), ((0, P - R), (0, 0)))

    boffs_h = (jnp.arange(B, dtype=jnp.int32) * Nh)[:, None]
    pool_abs = (pool_neigh_orders[: Nl * 7][None, :] + boffs_h).reshape(-1)
    pool_abs = jnp.pad(pool_abs, (0, (P - R) * 7))

    boffs_l = (jnp.arange(B, dtype=jnp.int32) * Nl)[:, None]
    k_off = jnp.tile(jnp.arange(7, dtype=jnp.int32), Nl)[None, :]
    ring_abs = ((neigh_orders[None, :] + boffs_l) * 7 + k_off).reshape(-1)
    ring_abs = jnp.pad(ring_abs, (0, (P - R) * 7))

    Wz = W1.reshape(CO, 7, C).transpose(2, 1, 0).reshape(C, 7 * CO)
    WaT = Wc[:, :CO].T
    WbT = Wc[:, CO:].T
    params = jnp.concatenate(
        [gamma[None, :], beta[None, :], bc[None, :],
         jnp.zeros((5, CO), jnp.float32)], axis=0)

    idx_shape = (_NW, P // (_V * _NW), _V * 7)
    xp = _gather_sum7_sc(xT, pool_abs.reshape(idx_shape), P)
    zp = _zp_matmul_tc(xp, Wz, 1.0 / 7.0)
    z = _gather_sum7_sc(zp.reshape(P * 7, CO),
                        ring_abs.reshape(idx_shape), P)
    stats = _stats_tc(z, R)
    yT = _final_tc(z, x1p, stats, params, WaT, WbT, R)
    return yT[:R].reshape(B, Nl, CO).transpose(0, 2, 1)
